# Initial kernel scaffold; baseline (speedup 1.0000x reference)
import functools

import jax
import jax.numpy as jnp
from jax import lax
from jax.experimental import pallas as pl
from jax.experimental.pallas import tpu as pltpu
from jax.experimental.pallas import tpu_sc as plsc

N = 10000
E = 320000
D = 128

NC = 2
NS = 16
NW = NC * NS
EPW = E // NW
CH = 80
NCHUNK = EPW // CH
RPT = N // NS
RSTAGE = 125


def _sc_aggregate_body(x_hbm, srcs_hbm, dsts_hbm, ws_hbm, out_hbm,
                       src_v, dst_v, w_v, rows_a, rows_b, stage, acc,
                       sem_a, sem_b):
    cid = lax.axis_index("c")
    sid = lax.axis_index("s")
    wid = sid * NC + cid

    zero16 = jnp.zeros((16,), jnp.float32)

    def _zero_row(i, _):
        for r in range(D // 16):
            stage[i, pl.ds(r * 16, 16)] = zero16
        return 0

    lax.fori_loop(0, RSTAGE, _zero_row, 0)
    for p in range(RPT // RSTAGE):
        pltpu.sync_copy(stage, acc.at[pl.ds(sid * RPT + p * RSTAGE, RSTAGE)])
    plsc.subcore_barrier()

    pltpu.sync_copy(srcs_hbm.at[wid], src_v)
    pltpu.sync_copy(dsts_hbm.at[wid], dst_v)
    pltpu.sync_copy(ws_hbm.at[wid], w_v)

    def _gather(c, rows, sem):
        return pltpu.async_copy(x_hbm.at[src_v.at[c]], rows, sem)

    def _wait(rows, sem):
        pltpu.make_async_copy(x_hbm.at[src_v.at[0]], rows, sem).wait()

    def _scale_and_scatter(c, rows):
        def _edge_group(g, _):
            for dj in range(4):
                j = g * 4 + dj
                w = w_v[c, j]
                for r in range(D // 16):
                    rows[j, pl.ds(r * 16, 16)] = rows[j, pl.ds(r * 16, 16)] * w
            return 0

        lax.fori_loop(0, CH // 4, _edge_group, 0)
        pltpu.sync_copy(rows, acc.at[dst_v.at[c]], add=True)

    _gather(0, rows_a, sem_a)

    def _pair(t, _):
        c = t * 2
        _wait(rows_a, sem_a)
        _gather(c + 1, rows_b, sem_b)
        _scale_and_scatter(c, rows_a)
        _wait(rows_b, sem_b)
        _gather(c + 2, rows_a, sem_a)
        _scale_and_scatter(c + 1, rows_b)
        return 0

    lax.fori_loop(0, (NCHUNK - 1) // 2, _pair, 0)
    _wait(rows_a, sem_a)
    _scale_and_scatter(NCHUNK - 1, rows_a)

    plsc.subcore_barrier()
    for p in range(RPT // RSTAGE):
        row0 = sid * RPT + p * RSTAGE
        pltpu.sync_copy(acc.at[pl.ds(row0, RSTAGE)], stage)
        pltpu.sync_copy(stage, out_hbm.at[cid, pl.ds(row0, RSTAGE)])


_sc_aggregate = functools.partial(
    pl.kernel,
    out_type=jax.ShapeDtypeStruct((NC, N, D), jnp.float32),
    mesh=plsc.VectorSubcoreMesh(core_axis_name="c", subcore_axis_name="s"),
    scratch_types=[
        pltpu.VMEM((NCHUNK, CH), jnp.int32),
        pltpu.VMEM((NCHUNK, CH), jnp.int32),
        pltpu.VMEM((NCHUNK, CH), jnp.float32),
        pltpu.VMEM((CH, D), jnp.float32),
        pltpu.VMEM((CH, D), jnp.float32),
        pltpu.VMEM((RSTAGE, D), jnp.float32),
        pltpu.VMEM_SHARED((N, D), jnp.float32),
        pltpu.SemaphoreType.DMA,
        pltpu.SemaphoreType.DMA,
    ],
)


def _sc_aggregate_call(x, src, dst, w):
    return _sc_aggregate(_sc_aggregate_body)(x, src, dst, w)


BM = 2000


def _matmul_body(p_ref, k_ref, b_ref, o_ref):
    s = p_ref[0] + p_ref[1]
    o_ref[...] = (
        jnp.dot(s, k_ref[...], preferred_element_type=jnp.float32)
        + b_ref[...]
    )


def _matmul(partial, k, bias2d):
    return pl.pallas_call(
        _matmul_body,
        out_shape=jax.ShapeDtypeStruct((N, D), jnp.float32),
        grid=(N // BM,),
        in_specs=[
            pl.BlockSpec((NC, BM, D), lambda i: (0, i, 0)),
            pl.BlockSpec((D, D), lambda i: (0, 0)),
            pl.BlockSpec((1, D), lambda i: (0, 0)),
        ],
        out_specs=pl.BlockSpec((BM, D), lambda i: (i, 0)),
    )(partial, k, bias2d)


@jax.jit
def kernel(x, edge_index, edge_weight, kernel, bias):
    src = edge_index[1].astype(jnp.int32).reshape(NW, NCHUNK, CH)
    dst = edge_index[0].astype(jnp.int32).reshape(NW, NCHUNK, CH)
    w = edge_weight.reshape(NW, NCHUNK, CH)
    partial = _sc_aggregate_call(x, src, dst, w)
    return _matmul(partial, kernel, bias.reshape(1, D))


# speedup vs baseline: 4.2712x; 4.2712x over previous
import functools

import jax
import jax.numpy as jnp
from jax import lax
from jax.experimental import pallas as pl
from jax.experimental.pallas import tpu as pltpu
from jax.experimental.pallas import tpu_sc as plsc

N = 10000
E = 320000
D = 128
HD = D // 2

NC = 2
NS = 16
EPW = E // NS
CH = 80
NCHUNK = EPW // CH
ACC_N = 10240
RPT = ACC_N // NS
RSTAGE = 128


def _sc_aggregate_body(xs_hbm, srcs_hbm, dsts_hbm, ws_hbm, out_hbm,
                       src_v, dst_v, w_v, rows_a, rows_b, stage, acc,
                       sem_a, sem_b):
    cid = lax.axis_index("c")
    sid = lax.axis_index("s")

    zero16 = jnp.zeros((16,), jnp.float32)

    def _zero_row(i, _):
        for r in range(HD // 16):
            stage[i, pl.ds(r * 16, 16)] = zero16
        return 0

    lax.fori_loop(0, RSTAGE, _zero_row, 0)
    for p in range(RPT // RSTAGE):
        pltpu.sync_copy(stage, acc.at[pl.ds(sid * RPT + p * RSTAGE, RSTAGE)])
    plsc.subcore_barrier()

    pltpu.sync_copy(srcs_hbm.at[cid, sid], src_v)
    pltpu.sync_copy(dsts_hbm.at[sid], dst_v)
    pltpu.sync_copy(ws_hbm.at[sid], w_v)

    def _gather(c, rows, sem):
        return pltpu.async_copy(xs_hbm.at[src_v.at[c]], rows, sem)

    def _wait(rows, sem):
        pltpu.make_async_copy(xs_hbm.at[src_v.at[0]], rows, sem).wait()

    def _scale_and_scatter(c, rows):
        def _edge_group(g, _):
            wv = w_v[c, pl.ds(g * 16, 16)]
            for l in range(16):
                j = g * 16 + l
                w = wv[l]
                for r in range(HD // 16):
                    rows[j, pl.ds(r * 16, 16)] = rows[j, pl.ds(r * 16, 16)] * w
            return 0

        lax.fori_loop(0, CH // 16, _edge_group, 0)
        pltpu.sync_copy(rows, acc.at[dst_v.at[c]], add=True)

    _gather(0, rows_a, sem_a)

    def _pair(t, _):
        c = t * 2
        _wait(rows_a, sem_a)
        _gather(c + 1, rows_b, sem_b)
        _scale_and_scatter(c, rows_a)
        _wait(rows_b, sem_b)
        _gather(c + 2, rows_a, sem_a)
        _scale_and_scatter(c + 1, rows_b)
        return 0

    lax.fori_loop(0, (NCHUNK - 2) // 2, _pair, 0)
    _wait(rows_a, sem_a)
    _gather(NCHUNK - 1, rows_b, sem_b)
    _scale_and_scatter(NCHUNK - 2, rows_a)
    _wait(rows_b, sem_b)
    _scale_and_scatter(NCHUNK - 1, rows_b)

    plsc.subcore_barrier()
    for p in range(RPT // RSTAGE):
        row0 = sid * RPT + p * RSTAGE
        pltpu.sync_copy(acc.at[pl.ds(row0, RSTAGE)], stage)
        pltpu.sync_copy(stage, out_hbm.at[cid, pl.ds(row0, RSTAGE)])


_sc_aggregate = pl.kernel(
    _sc_aggregate_body,
    out_type=jax.ShapeDtypeStruct((NC, ACC_N, HD), jnp.float32),
    mesh=plsc.VectorSubcoreMesh(core_axis_name="c", subcore_axis_name="s"),
    compiler_params=pltpu.CompilerParams(use_tc_tiling_on_sc=False),
    scratch_types=[
        pltpu.VMEM((NCHUNK, CH), jnp.int32),
        pltpu.VMEM((NCHUNK, CH), jnp.int32),
        pltpu.VMEM((NCHUNK, CH), jnp.float32),
        pltpu.VMEM((CH, HD), jnp.float32),
        pltpu.VMEM((CH, HD), jnp.float32),
        pltpu.VMEM((RSTAGE, HD), jnp.float32),
        pltpu.VMEM_SHARED((ACC_N, HD), jnp.float32),
        pltpu.SemaphoreType.DMA,
        pltpu.SemaphoreType.DMA,
    ],
)


BM = 2000


def _matmul_body(p_ref, k_ref, b_ref, o_ref):
    o_ref[...] = (
        jnp.dot(p_ref[0], k_ref[0:HD, :], preferred_element_type=jnp.float32)
        + jnp.dot(p_ref[1], k_ref[HD:D, :], preferred_element_type=jnp.float32)
        + b_ref[...]
    )


def _matmul(agg, k, bias2d):
    return pl.pallas_call(
        _matmul_body,
        out_shape=jax.ShapeDtypeStruct((N, D), jnp.float32),
        grid=(N // BM,),
        in_specs=[
            pl.BlockSpec((NC, BM, HD), lambda i: (0, i, 0)),
            pl.BlockSpec((D, D), lambda i: (0, 0)),
            pl.BlockSpec((1, D), lambda i: (0, 0)),
        ],
        out_specs=pl.BlockSpec((BM, D), lambda i: (i, 0)),
    )(agg, k, bias2d)


@jax.jit
def kernel(x, edge_index, edge_weight, kernel, bias):
    src = edge_index[1].astype(jnp.int32).reshape(NS, NCHUNK, CH)
    dst = edge_index[0].astype(jnp.int32).reshape(NS, NCHUNK, CH)
    srcs = jnp.stack([src, src + N])
    w = edge_weight.reshape(NS, NCHUNK, CH)
    xs = jnp.concatenate([x[:, :HD], x[:, HD:]], axis=0)
    agg = _sc_aggregate(xs, srcs, dst, w)
    return _matmul(agg, kernel, bias.reshape(1, D))
